# baseline (speedup 1.0000x reference)
import functools

import jax
import jax.numpy as jnp
from jax import lax
from jax.experimental import pallas as pl
from jax.experimental.pallas import tpu as pltpu
from jax.experimental.pallas import tpu_sc as plsc

N = 10000
E = 320000
DH = 128
NC = 2
NS = 16
NW = NC * NS
EPW = E // NW
CHUNK = 80
ROWS_PER_TILE = 624
ZROWS = 48

_MESH = plsc.VectorSubcoreMesh(core_axis_name="c", subcore_axis_name="s")
_SC_PARAMS = pltpu.CompilerParams(needs_layout_passes=False)


@functools.partial(
    pl.kernel,
    out_type=jax.ShapeDtypeStruct((NW * N,), jnp.float32),
    mesh=_MESH,
    compiler_params=_SC_PARAMS,
    scratch_types=[
        pltpu.VMEM((EPW,), jnp.int32),
        pltpu.VMEM((EPW,), jnp.float32),
        pltpu.VMEM((N,), jnp.float32),
    ],
)
def _deg_kernel(dst_hbm, w_hbm, out_hbm, dst_v, w_v, deg_v):
    cid = lax.axis_index("c")
    sid = lax.axis_index("s")
    wid = sid * NC + cid
    base = wid * EPW

    def zero_body(i, _):
        deg_v[pl.ds(i * 16, 16)] = jnp.zeros((16,), jnp.float32)
        return 0

    lax.fori_loop(0, N // 16, zero_body, 0)
    pltpu.sync_copy(dst_hbm.at[pl.ds(base, EPW)], dst_v)
    pltpu.sync_copy(w_hbm.at[pl.ds(base, EPW)], w_v)

    def add_body(i, _):
        d = dst_v[pl.ds(i * 16, 16)]
        w = w_v[pl.ds(i * 16, 16)]
        plsc.addupdate_scatter(deg_v, [d], w)
        return 0

    lax.fori_loop(0, EPW // 16, add_body, 0)
    pltpu.sync_copy(deg_v, out_hbm.at[pl.ds(wid * N, N)])


def _pre_body(degpart_ref, x_ref, wg_ref, xw2_ref, dinv_ref):
    deg = jnp.sum(degpart_ref[...], axis=0) + 1.0
    dinv = lax.rsqrt(deg)
    dinv_ref[...] = dinv
    xw = jnp.dot(x_ref[...], wg_ref[...], preferred_element_type=jnp.float32)
    xw2_ref[...] = xw * dinv[:, None]


_pre_call = pl.pallas_call(
    _pre_body,
    out_shape=(
        jax.ShapeDtypeStruct((N, DH), jnp.float32),
        jax.ShapeDtypeStruct((N,), jnp.float32),
    ),
)


NCHUNKS = EPW // CHUNK


@functools.partial(
    pl.kernel,
    out_type=jax.ShapeDtypeStruct((NC, N, DH), jnp.float32),
    mesh=_MESH,
    compiler_params=_SC_PARAMS,
    scratch_types=[
        pltpu.VMEM((N,), jnp.float32),
        pltpu.VMEM((CHUNK,), jnp.int32),
        pltpu.VMEM((CHUNK,), jnp.int32),
        pltpu.VMEM((CHUNK,), jnp.int32),
        pltpu.VMEM((CHUNK,), jnp.int32),
        pltpu.VMEM((CHUNK,), jnp.int32),
        pltpu.VMEM((CHUNK,), jnp.int32),
        pltpu.VMEM((CHUNK,), jnp.float32),
        pltpu.VMEM((CHUNK,), jnp.float32),
        pltpu.VMEM((CHUNK,), jnp.float32),
        pltpu.VMEM((CHUNK,), jnp.float32),
        pltpu.VMEM((CHUNK, DH), jnp.float32),
        pltpu.VMEM((CHUNK, DH), jnp.float32),
        pltpu.VMEM((ZROWS, DH), jnp.float32),
        pltpu.VMEM_SHARED((N, DH), jnp.float32),
        pltpu.SemaphoreType.DMA,
        pltpu.SemaphoreType.DMA,
        pltpu.SemaphoreType.DMA,
        pltpu.SemaphoreType.DMA,
        pltpu.SemaphoreType.DMA,
        pltpu.SemaphoreType.DMA,
        pltpu.SemaphoreType.DMA,
        pltpu.SemaphoreType.DMA,
    ],
)
def _msg_kernel(src_hbm, dst_hbm, w_hbm, xw2_hbm, dinv_hbm, out_hbm,
                dinv_v, srcA, srcB, dstA, dstB, dsiA, dsiB, wA, wB, scA, scB,
                rowsA, rowsB, zb_v, acc_sh,
                gsA, gsB, lsA, lsB, srA, srB, ssA, ssB):
    cid = lax.axis_index("c")
    sid = lax.axis_index("s")
    ebase = cid * (E // NC) + sid * EPW

    def load_src(i, src_ref, srcsem):
        pltpu.async_copy(src_hbm.at[pl.ds(ebase + i * CHUNK, CHUNK)], src_ref, srcsem)

    def load_dw(i, dst_ref, w_ref, lsem):
        b = ebase + i * CHUNK
        pltpu.async_copy(dst_hbm.at[pl.ds(b, CHUNK)], dst_ref, lsem)
        pltpu.async_copy(w_hbm.at[pl.ds(b, CHUNK)], w_ref, lsem)

    def issue_gather(i, src_ref, rows_ref, srcsem, gsem):
        pltpu.make_async_copy(src_hbm.at[pl.ds(ebase + i * CHUNK, CHUNK)],
                              src_ref, srcsem).wait()
        pltpu.async_copy(xw2_hbm.at[src_ref], rows_ref, gsem)

    load_src(0, srcA, srA)
    load_dw(0, dstA, wA, lsA)
    load_src(1, srcB, srB)
    load_dw(1, dstB, wB, lsB)
    issue_gather(0, srcA, rowsA, srA, gsA)
    issue_gather(1, srcB, rowsB, srB, gsB)

    def zb_body(r, _):
        for j in range(DH // 16):
            zb_v[r, pl.ds(j * 16, 16)] = jnp.zeros((16,), jnp.float32)
        return 0

    lax.fori_loop(0, ZROWS, zb_body, 0)
    for k in range(ROWS_PER_TILE // ZROWS):
        pltpu.sync_copy(zb_v, acc_sh.at[pl.ds(sid * ROWS_PER_TILE + k * ZROWS, ZROWS)])

    @pl.when(sid == NS - 1)
    def _zero_tail():
        pltpu.sync_copy(zb_v.at[pl.ds(0, N - NS * ROWS_PER_TILE)],
                        acc_sh.at[pl.ds(NS * ROWS_PER_TILE, N - NS * ROWS_PER_TILE)])

    pltpu.sync_copy(dinv_hbm, dinv_v)
    plsc.subcore_barrier()

    def process(i, src_ref, dst_ref, dsi_ref, w_ref, rows_ref, scale_ref,
                lsem, gsem, srcsem):
        b = ebase + i * CHUNK
        pltpu.make_async_copy(dst_hbm.at[pl.ds(b, CHUNK)], dst_ref, lsem).wait()
        pltpu.make_async_copy(w_hbm.at[pl.ds(b, CHUNK)], w_ref, lsem).wait()
        def scale_body(g, _):
            d = dst_ref[pl.ds(g * 16, 16)]
            dv = plsc.load_gather(dinv_v, [d])
            scale_ref[pl.ds(g * 16, 16)] = w_ref[pl.ds(g * 16, 16)] * dv
            return 0

        lax.fori_loop(0, CHUNK // 16, scale_body, 0)

        pltpu.make_async_copy(xw2_hbm.at[src_ref], rows_ref, gsem).wait()

        @pl.when(i + 2 < NCHUNKS)
        def _prefetch_src():
            load_src(i + 2, src_ref, srcsem)

        def row_body(it, _):
            for r in range(10):
                k = it * 10 + r
                s = plsc.load_gather(scale_ref, [jnp.full((16,), k, jnp.int32)])
                for j in range(DH // 16):
                    rows_ref[k, pl.ds(j * 16, 16)] = rows_ref[k, pl.ds(j * 16, 16)] * s
            return 0

        lax.fori_loop(0, CHUNK // 10, row_body, 0)

    def outer(o, _):
        i0 = 2 * o
        i1 = i0 + 1
        process(i0, srcA, dstA, dsiA, wA, rowsA, scA, lsA, gsA, srA)
        pltpu.async_copy(rowsA, acc_sh.at[dstA], ssA, add=True)
        process(i1, srcB, dstB, dsiB, wB, rowsB, scB, lsB, gsB, srB)
        pltpu.async_copy(rowsB, acc_sh.at[dstB], ssB, add=True)
        pltpu.make_async_copy(rowsA, acc_sh.at[dstA], ssA).wait()
        load_dw(i0 + 2, dstA, wA, lsA)
        issue_gather(i0 + 2, srcA, rowsA, srA, gsA)
        pltpu.make_async_copy(rowsB, acc_sh.at[dstB], ssB).wait()

        @pl.when(i1 + 2 < NCHUNKS)
        def _prefetch_dw_b():
            load_dw(i1 + 2, dstB, wB, lsB)
            issue_gather(i1 + 2, srcB, rowsB, srB, gsB)

        return 0

    lax.fori_loop(0, (NCHUNKS - 1) // 2, outer, 0)
    process(NCHUNKS - 1, srcA, dstA, dsiA, wA, rowsA, scA, lsA, gsA, srA)
    pltpu.sync_copy(rowsA, acc_sh.at[dstA], add=True)
    plsc.subcore_barrier()
    pltpu.sync_copy(acc_sh.at[pl.ds(sid * ROWS_PER_TILE, ROWS_PER_TILE)],
                    out_hbm.at[cid, pl.ds(sid * ROWS_PER_TILE, ROWS_PER_TILE)])

    @pl.when(sid == NS - 1)
    def _write_tail():
        pltpu.sync_copy(acc_sh.at[pl.ds(NS * ROWS_PER_TILE, N - NS * ROWS_PER_TILE)],
                        out_hbm.at[cid, pl.ds(NS * ROWS_PER_TILE, N - NS * ROWS_PER_TILE)])


def _gru_body(part_ref, xw2_ref, dinv_ref, h_ref, wih_ref, whh_ref,
              bih_ref, bhh_ref, bg_ref, out_ref):
    dinv = dinv_ref[...]
    spatial = (part_ref[0] + part_ref[1]
               + xw2_ref[...] * dinv[:, None]
               + bg_ref[...])
    gi = lax.dot_general(spatial, wih_ref[...], (((1,), (1,)), ((), ())),
                         preferred_element_type=jnp.float32) + bih_ref[...]
    gh = lax.dot_general(h_ref[...], whh_ref[...], (((1,), (1,)), ((), ())),
                         preferred_element_type=jnp.float32) + bhh_ref[...]
    i_r, i_z, i_n = gi[:, :DH], gi[:, DH:2 * DH], gi[:, 2 * DH:]
    h_r, h_z, h_n = gh[:, :DH], gh[:, DH:2 * DH], gh[:, 2 * DH:]
    r = jax.nn.sigmoid(i_r + h_r)
    z = jax.nn.sigmoid(i_z + h_z)
    n = jnp.tanh(i_n + r * h_n)
    out_ref[...] = (1.0 - z) * n + z * h_ref[...]


_gru_call = pl.pallas_call(
    _gru_body,
    out_shape=jax.ShapeDtypeStruct((N, DH), jnp.float32),
)


def kernel(x, h, edge_index, edge_weight, W_gcn, b_gcn, W_ih, W_hh, b_ih, b_hh):
    src = edge_index[0]
    dst = edge_index[1]
    degpart = _deg_kernel(dst, edge_weight).reshape(NW, N)
    xw2, dinv = _pre_call(degpart, x, W_gcn)
    part = _msg_kernel(src, dst, edge_weight, xw2, dinv)
    return _gru_call(part, xw2, dinv, h, W_ih, W_hh, b_ih, b_hh, b_gcn)


# speedup vs baseline: 1.9551x; 1.9551x over previous
import functools

import jax
import jax.numpy as jnp
from jax import lax
from jax.experimental import pallas as pl
from jax.experimental.pallas import tpu as pltpu
from jax.experimental.pallas import tpu_sc as plsc

N = 10000
E = 320000
DH = 128
NC = 2
NS = 16
NW = NC * NS
EPW = E // NW
CHUNK = 80
ROWS_PER_TILE = 624
ZROWS = 48

_MESH = plsc.VectorSubcoreMesh(core_axis_name="c", subcore_axis_name="s")
_SC_PARAMS = pltpu.CompilerParams(needs_layout_passes=False)


@functools.partial(
    pl.kernel,
    out_type=jax.ShapeDtypeStruct((NW * N,), jnp.float32),
    mesh=_MESH,
    compiler_params=_SC_PARAMS,
    scratch_types=[
        pltpu.VMEM((EPW,), jnp.int32),
        pltpu.VMEM((EPW,), jnp.float32),
        pltpu.VMEM((N,), jnp.float32),
    ],
)
def _deg_kernel(dst_hbm, w_hbm, out_hbm, dst_v, w_v, deg_v):
    cid = lax.axis_index("c")
    sid = lax.axis_index("s")
    wid = sid * NC + cid
    base = wid * EPW

    def zero_body(i, _):
        deg_v[pl.ds(i * 16, 16)] = jnp.zeros((16,), jnp.float32)
        return 0

    lax.fori_loop(0, N // 16, zero_body, 0)
    pltpu.sync_copy(dst_hbm.at[pl.ds(base, EPW)], dst_v)
    pltpu.sync_copy(w_hbm.at[pl.ds(base, EPW)], w_v)

    def add_body(i, _):
        d = dst_v[pl.ds(i * 16, 16)]
        w = w_v[pl.ds(i * 16, 16)]
        plsc.addupdate_scatter(deg_v, [d], w)
        return 0

    lax.fori_loop(0, EPW // 16, add_body, 0)
    pltpu.sync_copy(deg_v, out_hbm.at[pl.ds(wid * N, N)])


def _pre_body(degpart_ref, x_ref, wg_ref, xw2_ref, dinv_ref):
    deg = jnp.sum(degpart_ref[...], axis=0) + 1.0
    dinv = lax.rsqrt(deg)
    dinv_ref[...] = dinv
    xw = jnp.dot(x_ref[...], wg_ref[...], preferred_element_type=jnp.float32)
    xw2_ref[...] = xw * dinv[:, None]


_pre_call = pl.pallas_call(
    _pre_body,
    out_shape=(
        jax.ShapeDtypeStruct((N, DH), jnp.float32),
        jax.ShapeDtypeStruct((N,), jnp.float32),
    ),
)


NCHUNKS = EPW // CHUNK


@functools.partial(
    pl.kernel,
    out_type=jax.ShapeDtypeStruct((NC, N, DH), jnp.float32),
    mesh=_MESH,
    compiler_params=_SC_PARAMS,
    scratch_types=[
        pltpu.VMEM((N,), jnp.float32),
        pltpu.VMEM((CHUNK,), jnp.int32),
        pltpu.VMEM((CHUNK,), jnp.int32),
        pltpu.VMEM((CHUNK,), jnp.int32),
        pltpu.VMEM((CHUNK,), jnp.int32),
        pltpu.VMEM((CHUNK,), jnp.int32),
        pltpu.VMEM((CHUNK,), jnp.int32),
        pltpu.VMEM((CHUNK,), jnp.float32),
        pltpu.VMEM((CHUNK,), jnp.float32),
        pltpu.VMEM((CHUNK,), jnp.float32),
        pltpu.VMEM((CHUNK,), jnp.float32),
        pltpu.VMEM((CHUNK, DH), jnp.float32),
        pltpu.VMEM((CHUNK, DH), jnp.float32),
        pltpu.VMEM((ZROWS, DH), jnp.float32),
        pltpu.VMEM_SHARED((N, DH), jnp.float32),
        pltpu.SemaphoreType.DMA,
        pltpu.SemaphoreType.DMA,
        pltpu.SemaphoreType.DMA,
        pltpu.SemaphoreType.DMA,
        pltpu.SemaphoreType.DMA,
        pltpu.SemaphoreType.DMA,
        pltpu.SemaphoreType.DMA,
        pltpu.SemaphoreType.DMA,
    ],
)
def _msg_kernel(src_hbm, dst_hbm, w_hbm, xw2_hbm, dinv_hbm, out_hbm,
                dinv_v, srcA, srcB, dstA, dstB, dsiA, dsiB, wA, wB, scA, scB,
                rowsA, rowsB, zb_v, acc_sh,
                gsA, gsB, lsA, lsB, srA, srB, ssA, ssB):
    cid = lax.axis_index("c")
    sid = lax.axis_index("s")
    ebase = cid * (E // NC) + sid * EPW

    def load_src(i, src_ref, srcsem):
        pltpu.async_copy(src_hbm.at[pl.ds(ebase + i * CHUNK, CHUNK)], src_ref, srcsem)

    def load_dw(i, dst_ref, w_ref, lsem):
        b = ebase + i * CHUNK
        pltpu.async_copy(dst_hbm.at[pl.ds(b, CHUNK)], dst_ref, lsem)
        pltpu.async_copy(w_hbm.at[pl.ds(b, CHUNK)], w_ref, lsem)

    def issue_gather(i, src_ref, rows_ref, srcsem, gsem):
        pltpu.make_async_copy(src_hbm.at[pl.ds(ebase + i * CHUNK, CHUNK)],
                              src_ref, srcsem).wait()
        pltpu.async_copy(xw2_hbm.at[src_ref], rows_ref, gsem)

    load_src(0, srcA, srA)
    load_dw(0, dstA, wA, lsA)
    load_src(1, srcB, srB)
    load_dw(1, dstB, wB, lsB)
    issue_gather(0, srcA, rowsA, srA, gsA)
    issue_gather(1, srcB, rowsB, srB, gsB)

    def zb_body(r, _):
        for j in range(DH // 16):
            zb_v[r, pl.ds(j * 16, 16)] = jnp.zeros((16,), jnp.float32)
        return 0

    lax.fori_loop(0, ZROWS, zb_body, 0)
    for k in range(ROWS_PER_TILE // ZROWS):
        pltpu.sync_copy(zb_v, acc_sh.at[pl.ds(sid * ROWS_PER_TILE + k * ZROWS, ZROWS)])

    @pl.when(sid == NS - 1)
    def _zero_tail():
        pltpu.sync_copy(zb_v.at[pl.ds(0, N - NS * ROWS_PER_TILE)],
                        acc_sh.at[pl.ds(NS * ROWS_PER_TILE, N - NS * ROWS_PER_TILE)])

    pltpu.sync_copy(dinv_hbm, dinv_v)
    plsc.subcore_barrier()

    def process(i, src_ref, dst_ref, dsi_ref, w_ref, rows_ref, scale_ref,
                lsem, gsem, srcsem):
        b = ebase + i * CHUNK
        pltpu.make_async_copy(dst_hbm.at[pl.ds(b, CHUNK)], dst_ref, lsem).wait()
        pltpu.make_async_copy(w_hbm.at[pl.ds(b, CHUNK)], w_ref, lsem).wait()
        def scale_body(g, _):
            d = dst_ref[pl.ds(g * 16, 16)]
            dv = plsc.load_gather(dinv_v, [d])
            scale_ref[pl.ds(g * 16, 16)] = w_ref[pl.ds(g * 16, 16)] * dv
            return 0

        lax.fori_loop(0, CHUNK // 16, scale_body, 0)

        pltpu.make_async_copy(xw2_hbm.at[src_ref], rows_ref, gsem).wait()

        @pl.when(i + 2 < NCHUNKS)
        def _prefetch_src():
            load_src(i + 2, src_ref, srcsem)

        def row_body(it, _):
            for r in range(5):
                k = it * 5 + r
                s = plsc.load_gather(scale_ref, [jnp.full((16,), k, jnp.int32)])
                for j in range(DH // 16):
                    rows_ref[k, pl.ds(j * 16, 16)] = rows_ref[k, pl.ds(j * 16, 16)] * s
            return 0

        lax.fori_loop(0, CHUNK // 5, row_body, 0)

    def outer(o, _):
        i0 = 2 * o
        i1 = i0 + 1
        process(i0, srcA, dstA, dsiA, wA, rowsA, scA, lsA, gsA, srA)
        pltpu.async_copy(rowsA, acc_sh.at[dstA], ssA, add=True)
        process(i1, srcB, dstB, dsiB, wB, rowsB, scB, lsB, gsB, srB)
        pltpu.async_copy(rowsB, acc_sh.at[dstB], ssB, add=True)
        pltpu.make_async_copy(rowsA, acc_sh.at[dstA], ssA).wait()
        load_dw(i0 + 2, dstA, wA, lsA)
        issue_gather(i0 + 2, srcA, rowsA, srA, gsA)
        pltpu.make_async_copy(rowsB, acc_sh.at[dstB], ssB).wait()

        @pl.when(i1 + 2 < NCHUNKS)
        def _prefetch_dw_b():
            load_dw(i1 + 2, dstB, wB, lsB)
            issue_gather(i1 + 2, srcB, rowsB, srB, gsB)

        return 0

    lax.fori_loop(0, (NCHUNKS - 1) // 2, outer, 0)
    process(NCHUNKS - 1, srcA, dstA, dsiA, wA, rowsA, scA, lsA, gsA, srA)
    pltpu.sync_copy(rowsA, acc_sh.at[dstA], add=True)
    plsc.subcore_barrier()
    pltpu.sync_copy(acc_sh.at[pl.ds(sid * ROWS_PER_TILE, ROWS_PER_TILE)],
                    out_hbm.at[cid, pl.ds(sid * ROWS_PER_TILE, ROWS_PER_TILE)])

    @pl.when(sid == NS - 1)
    def _write_tail():
        pltpu.sync_copy(acc_sh.at[pl.ds(NS * ROWS_PER_TILE, N - NS * ROWS_PER_TILE)],
                        out_hbm.at[cid, pl.ds(NS * ROWS_PER_TILE, N - NS * ROWS_PER_TILE)])


def _gru_body(part_ref, xw2_ref, dinv_ref, h_ref, wih_ref, whh_ref,
              bih_ref, bhh_ref, bg_ref, out_ref):
    dinv = dinv_ref[...]
    spatial = (part_ref[0] + part_ref[1]
               + xw2_ref[...] * dinv[:, None]
               + bg_ref[...])
    gi = lax.dot_general(spatial, wih_ref[...], (((1,), (1,)), ((), ())),
                         preferred_element_type=jnp.float32) + bih_ref[...]
    gh = lax.dot_general(h_ref[...], whh_ref[...], (((1,), (1,)), ((), ())),
                         preferred_element_type=jnp.float32) + bhh_ref[...]
    i_r, i_z, i_n = gi[:, :DH], gi[:, DH:2 * DH], gi[:, 2 * DH:]
    h_r, h_z, h_n = gh[:, :DH], gh[:, DH:2 * DH], gh[:, 2 * DH:]
    r = jax.nn.sigmoid(i_r + h_r)
    z = jax.nn.sigmoid(i_z + h_z)
    n = jnp.tanh(i_n + r * h_n)
    out_ref[...] = (1.0 - z) * n + z * h_ref[...]


_gru_call = pl.pallas_call(
    _gru_body,
    out_shape=jax.ShapeDtypeStruct((N, DH), jnp.float32),
)


def kernel(x, h, edge_index, edge_weight, W_gcn, b_gcn, W_ih, W_hh, b_ih, b_hh):
    src = edge_index[0]
    dst = edge_index[1]
    degpart = _deg_kernel(dst, edge_weight).reshape(NW, N)
    xw2, dinv = _pre_call(degpart, x, W_gcn)
    part = _msg_kernel(src, dst, edge_weight, xw2, dinv)
    return _gru_call(part, xw2, dinv, h, W_ih, W_hh, b_ih, b_hh, b_gcn)
